# baseline (speedup 1.0000x reference)
import functools

import jax
import jax.numpy as jnp
from jax import lax
from jax.experimental import pallas as pl
from jax.experimental.pallas import tpu as pltpu
from jax.experimental.pallas import tpu_sc as plsc

SEQ = 200
BATCH = 4096
DIM = 64
OUT_DIM = 128
NC = 2
NS = 16
NW = NC * NS
BPW = BATCH // NW
C0 = 128
C1 = SEQ - C0


def _sc_pool_body(text_hbm, table_hbm, out_hbm, raw_v, idx_v, rows_v, sums_v,
                  sems):
    wid = lax.axis_index("s") * NC + lax.axis_index("c")
    base = wid * BPW
    pltpu.sync_copy(text_hbm.at[:, pl.ds(base, BPW)], raw_v)

    lanes = lax.iota(jnp.int32, 16)

    def trans_row(s, carry):
        for c in range(BPW // 16):
            dest = (lanes + (c * 16)) * SEQ + s
            plsc.store_scatter(idx_v, [dest], raw_v[s, c * 16:(c + 1) * 16])
        return carry

    lax.fori_loop(0, SEQ, trans_row, 0, unroll=2)

    def gather_col(j, buf):
        pltpu.async_copy(
            table_hbm.at[idx_v.at[pl.ds(j * SEQ, C0)]],
            rows_v.at[buf, pl.ds(0, C0)], sems.at[buf])
        pltpu.async_copy(
            table_hbm.at[idx_v.at[pl.ds(j * SEQ + C0, C1)]],
            rows_v.at[buf, pl.ds(C0, C1)], sems.at[buf])

    def wait_col(j, buf):
        pltpu.make_async_copy(
            table_hbm.at[idx_v.at[pl.ds(j * SEQ, C0)]],
            rows_v.at[buf, pl.ds(0, C0)], sems.at[buf]).wait()
        pltpu.make_async_copy(
            table_hbm.at[idx_v.at[pl.ds(j * SEQ + C0, C1)]],
            rows_v.at[buf, pl.ds(C0, C1)], sems.at[buf]).wait()

    def accum_col(j, buf):
        def srow(s, acc):
            a0, a1, a2, a3 = acc
            return (a0 + rows_v[buf, s, 0:16], a1 + rows_v[buf, s, 16:32],
                    a2 + rows_v[buf, s, 32:48], a3 + rows_v[buf, s, 48:64])

        z = jnp.zeros((16,), jnp.float32)
        a0, a1, a2, a3 = lax.fori_loop(0, SEQ, srow, (z, z, z, z),
                                       unroll=8)
        scale = jnp.float32(1.0 / SEQ)
        sums_v[j, 0:16] = a0 * scale
        sums_v[j, 16:32] = a1 * scale
        sums_v[j, 32:48] = a2 * scale
        sums_v[j, 48:64] = a3 * scale

    gather_col(0, 0)

    def pair(i, carry):
        j = 2 * i
        gather_col(j + 1, 1)
        wait_col(j, 0)
        accum_col(j, 0)

        @pl.when(j + 2 < BPW)
        def _():
            gather_col(j + 2, 0)

        wait_col(j + 1, 1)
        accum_col(j + 1, 1)
        return carry

    lax.fori_loop(0, BPW // 2, pair, 0)
    pltpu.sync_copy(sums_v, out_hbm.at[pl.ds(base, BPW)])


@jax.jit
def _sc_pool(text, table):
    mesh = plsc.VectorSubcoreMesh(core_axis_name="c", subcore_axis_name="s")
    return pl.kernel(
        _sc_pool_body,
        out_type=jax.ShapeDtypeStruct((BATCH, DIM), jnp.float32),
        mesh=mesh,
        scratch_types=[
            pltpu.VMEM((SEQ, BPW), jnp.int32),
            pltpu.VMEM((BPW * SEQ,), jnp.int32),
            pltpu.VMEM((2, SEQ, DIM), jnp.float32),
            pltpu.VMEM((BPW, DIM), jnp.float32),
            pltpu.SemaphoreType.DMA((2,)),
        ],
        compiler_params=pltpu.CompilerParams(use_tc_tiling_on_sc=False,
                                             needs_layout_passes=False),
    )(text, table)


def _tc_fc_body(x_ref, w_ref, b_ref, o_ref):
    o_ref[...] = lax.dot_general(
        x_ref[...], w_ref[...], (((1,), (1,)), ((), ())),
        preferred_element_type=jnp.float32) + b_ref[...]


@jax.jit
def _tc_fc(pooled, W, b2d):
    return pl.pallas_call(
        _tc_fc_body,
        out_shape=jax.ShapeDtypeStruct((BATCH, OUT_DIM), jnp.float32),
    )(pooled, W, b2d)


def kernel(text, emb_table, W, b):
    pooled = _sc_pool(text.astype(jnp.int32), emb_table)
    return _tc_fc(pooled, W, b.reshape(1, OUT_DIM))


# speedup vs baseline: 1.0128x; 1.0128x over previous
import functools

import jax
import jax.numpy as jnp
from jax import lax
from jax.experimental import pallas as pl
from jax.experimental.pallas import tpu as pltpu
from jax.experimental.pallas import tpu_sc as plsc

SEQ = 200
BATCH = 4096
DIM = 64
OUT_DIM = 128
NC = 2
NS = 16
NW = NC * NS
BPW = BATCH // NW
NTR = SEQ // 8
C0 = 128
C1 = SEQ - C0


def _sc_transpose_body(text_hbm, out_hbm, raw_v, idxT_v):
    wid = lax.axis_index("s") * NC + lax.axis_index("c")
    base = wid * BPW
    for tr in range(NTR):
        pltpu.sync_copy(text_hbm.at[pl.ds(tr * 8, 8), pl.ds(base, BPW)],
                        raw_v.at[tr])
    lanes = lax.iota(jnp.int32, 16)

    def trans_tile(tr, carry):
        s0 = tr * 8
        for r in range(8):
            for c in range(BPW // 16):
                dest = (lanes + (c * 16)) * SEQ + (s0 + r)
                plsc.store_scatter(idxT_v, [dest],
                                   raw_v[tr, r, c * 16:(c + 1) * 16])
        return carry

    lax.fori_loop(0, NTR, trans_tile, 0)
    pltpu.sync_copy(idxT_v, out_hbm.at[pl.ds(base * SEQ, BPW * SEQ)])


@jax.jit
def _sc_transpose(text):
    mesh = plsc.VectorSubcoreMesh(core_axis_name="c", subcore_axis_name="s")
    return pl.kernel(
        _sc_transpose_body,
        out_type=jax.ShapeDtypeStruct((BATCH * SEQ,), jnp.int32),
        mesh=mesh,
        scratch_types=[
            pltpu.VMEM((NTR, 8, BPW), jnp.int32),
            pltpu.VMEM((BPW * SEQ,), jnp.int32),
        ],
        compiler_params=pltpu.CompilerParams(use_tc_tiling_on_sc=True,
                                             needs_layout_passes=False),
    )(text)


def _sc_pool_body(idxT_hbm, table_hbm, out_hbm, idx_v, rows_v, sums_v, sems):
    wid = lax.axis_index("s") * NC + lax.axis_index("c")
    base = wid * BPW
    pltpu.sync_copy(idxT_hbm.at[pl.ds(base * SEQ, BPW * SEQ)], idx_v)

    def gather_col(j, buf):
        pltpu.async_copy(
            table_hbm.at[idx_v.at[pl.ds(j * SEQ, C0)]],
            rows_v.at[buf, pl.ds(0, C0)], sems.at[buf])
        pltpu.async_copy(
            table_hbm.at[idx_v.at[pl.ds(j * SEQ + C0, C1)]],
            rows_v.at[buf, pl.ds(C0, C1)], sems.at[buf])

    def wait_col(j, buf):
        pltpu.make_async_copy(
            table_hbm.at[idx_v.at[pl.ds(j * SEQ, C0)]],
            rows_v.at[buf, pl.ds(0, C0)], sems.at[buf]).wait()
        pltpu.make_async_copy(
            table_hbm.at[idx_v.at[pl.ds(j * SEQ + C0, C1)]],
            rows_v.at[buf, pl.ds(C0, C1)], sems.at[buf]).wait()

    def accum_col(j, buf):
        def srow(s, acc):
            a0, a1, a2, a3 = acc
            return (a0 + rows_v[buf, s, 0:16], a1 + rows_v[buf, s, 16:32],
                    a2 + rows_v[buf, s, 32:48], a3 + rows_v[buf, s, 48:64])

        z = jnp.zeros((16,), jnp.float32)
        a0, a1, a2, a3 = lax.fori_loop(0, SEQ, srow, (z, z, z, z),
                                       unroll=8)
        scale = jnp.float32(1.0 / SEQ)
        sums_v[j, 0:16] = a0 * scale
        sums_v[j, 16:32] = a1 * scale
        sums_v[j, 32:48] = a2 * scale
        sums_v[j, 48:64] = a3 * scale

    gather_col(0, 0)

    def pair(i, carry):
        j = 2 * i
        gather_col(j + 1, 1)
        wait_col(j, 0)
        accum_col(j, 0)

        @pl.when(j + 2 < BPW)
        def _():
            gather_col(j + 2, 0)

        wait_col(j + 1, 1)
        accum_col(j + 1, 1)
        return carry

    lax.fori_loop(0, BPW // 2, pair, 0)
    pltpu.sync_copy(sums_v, out_hbm.at[pl.ds(base, BPW)])


@jax.jit
def _sc_pool(idxT, table):
    mesh = plsc.VectorSubcoreMesh(core_axis_name="c", subcore_axis_name="s")
    return pl.kernel(
        _sc_pool_body,
        out_type=jax.ShapeDtypeStruct((BATCH, DIM), jnp.float32),
        mesh=mesh,
        scratch_types=[
            pltpu.VMEM((BPW * SEQ,), jnp.int32),
            pltpu.VMEM((2, SEQ, DIM), jnp.float32),
            pltpu.VMEM((BPW, DIM), jnp.float32),
            pltpu.SemaphoreType.DMA((2,)),
        ],
        compiler_params=pltpu.CompilerParams(use_tc_tiling_on_sc=False,
                                             needs_layout_passes=False),
    )(idxT, table)


def _tc_fc_body(x_ref, w_ref, b_ref, o_ref):
    o_ref[...] = lax.dot_general(
        x_ref[...], w_ref[...], (((1,), (1,)), ((), ())),
        preferred_element_type=jnp.float32) + b_ref[...]


@jax.jit
def _tc_fc(pooled, W, b2d):
    return pl.pallas_call(
        _tc_fc_body,
        out_shape=jax.ShapeDtypeStruct((BATCH, OUT_DIM), jnp.float32),
    )(pooled, W, b2d)


def kernel(text, emb_table, W, b):
    idxT = _sc_transpose(text.astype(jnp.int32))
    pooled = _sc_pool(idxT, emb_table)
    return _tc_fc(pooled, W, b.reshape(1, OUT_DIM))
